# baseline (speedup 1.0000x reference)
import functools

import jax
import jax.numpy as jnp
from jax import lax
from jax.experimental import pallas as pl
from jax.experimental.pallas import tpu as pltpu
from jax.experimental.pallas import tpu_sc as plsc

_N = 10000
_E = 320000
_D = 128
_EPS = 1e-5

_NC = 2
_NS = 16
_NW = _NC * _NS
_EPT = _E // _NW
_CH = 80
_NCHUNK = _EPT // _CH
_GR = _CH // 16

_NPH = 4
_PH = 2560
_ACC = _PH + _NS
_RZ = _ACC // _NS
_RW = _PH // _NS
_NP2 = _NPH * _PH

_mesh = plsc.VectorSubcoreMesh(core_axis_name="c", subcore_axis_name="s")


@functools.partial(
    pl.kernel,
    out_type=jax.ShapeDtypeStruct((_NC, _NP2, _D), jnp.float32),
    mesh=_mesh,
    scratch_types=[
        pltpu.VMEM((_NCHUNK, _CH), jnp.int32),
        pltpu.VMEM((_NCHUNK, _CH), jnp.int32),
        pltpu.VMEM((_NCHUNK, _CH), jnp.int32),
        pltpu.VMEM((_NCHUNK, _CH), jnp.int32),
        pltpu.VMEM((_CH, _D), jnp.float32),
        pltpu.VMEM((_RZ, _D), jnp.float32),
        pltpu.VMEM_SHARED((_ACC, _D), jnp.float32),
        pltpu.SemaphoreType.DMA,
        pltpu.SemaphoreType.DMA,
    ],
)
def _scatter_kernel(hp_hbm, src_hbm, dst_hbm, out_hbm,
                    src_v, dst_v, srcp_v, dstp_v, rows_v, zbuf_v, acc_sh,
                    gsem, ssem):
    c = lax.axis_index("c")
    s = lax.axis_index("s")
    wid = c * _NS + s
    dump = _PH + s

    pltpu.sync_copy(src_hbm.at[wid], src_v)
    pltpu.sync_copy(dst_hbm.at[wid], dst_v)

    def _zbufz(r, _):
        for k in range(_D // 16):
            zbuf_v[r, pl.ds(k * 16, 16)] = jnp.zeros((16,), jnp.float32)
        return 0

    for phase in range(_NPH):
        lo = phase * _PH
        lax.fori_loop(0, _RZ, _zbufz, 0)
        pltpu.sync_copy(zbuf_v, acc_sh.at[pl.ds(s * _RZ, _RZ)])

        dump_vec = jnp.zeros((16,), jnp.int32) + dump
        lo_i = jnp.int32(lo)
        hi_i = jnp.int32(lo + _PH)

        def _clamp(r, _):
            for k in range(_GR):
                d16 = dst_v[r, pl.ds(k * 16, 16)]
                s16 = src_v[r, pl.ds(k * 16, 16)]
                ok = (d16 >= lo_i) & (d16 < hi_i)
                dstp_v[r, pl.ds(k * 16, 16)] = jnp.where(ok, d16 - lo_i,
                                                         dump_vec)
                srcp_v[r, pl.ds(k * 16, 16)] = jnp.where(
                    ok, s16, jnp.zeros((16,), jnp.int32))
            return 0

        lax.fori_loop(0, _NCHUNK, _clamp, 0)
        plsc.subcore_barrier()

        def _edge_chunk(j, _):
            pltpu.async_copy(hp_hbm.at[srcp_v.at[j]], rows_v, gsem).wait()
            pltpu.async_copy(rows_v, acc_sh.at[dstp_v.at[j]], ssem,
                             add=True).wait()
            return 0

        lax.fori_loop(0, _NCHUNK, _edge_chunk, 0)
        plsc.subcore_barrier()

        pltpu.sync_copy(acc_sh.at[pl.ds(s * _RW, _RW)], zbuf_v.at[pl.ds(0, _RW)])
        pltpu.sync_copy(zbuf_v.at[pl.ds(0, _RW)],
                        out_hbm.at[c, pl.ds(lo + s * _RW, _RW)])
        plsc.subcore_barrier()


def _prep_body(degp_ref, x_ref, w1_ref, dinv_ref, hp_ref):
    deg = degp_ref[0, 0:_N, 0:1] + degp_ref[1, 0:_N, 0:1] + 1.0
    dinv = lax.rsqrt(deg)
    dinv_ref[...] = dinv
    h1 = jnp.dot(x_ref[...], w1_ref[...], preferred_element_type=jnp.float32)
    hp_ref[...] = dinv * h1


_prep_call = pl.pallas_call(
    _prep_body,
    out_shape=[
        jax.ShapeDtypeStruct((_N, 1), jnp.float32),
        jax.ShapeDtypeStruct((_N, _D), jnp.float32),
    ],
)


def _combine_body(has_res, has_next, *refs):
    if has_res and has_next:
        (s_ref, hp_ref, dinv_ref, b_ref, g_ref, be_ref,
         hprev_ref, wn_ref, y_ref, hn_ref) = refs
    elif has_next:
        (s_ref, hp_ref, dinv_ref, b_ref, g_ref, be_ref,
         wn_ref, y_ref, hn_ref) = refs
    else:
        (s_ref, hp_ref, dinv_ref, b_ref, g_ref, be_ref,
         hprev_ref, y_ref) = refs

    dinv = dinv_ref[...]
    agg = s_ref[0, 0:_N, :] + s_ref[1, 0:_N, :] + hp_ref[...]
    z = dinv * agg + b_ref[...]
    mu = jnp.mean(z, axis=0, keepdims=True)
    zc = z - mu
    var = jnp.mean(zc * zc, axis=0, keepdims=True)
    y = g_ref[...] * zc * lax.rsqrt(var + _EPS) + be_ref[...]
    y = jnp.maximum(y, 0.0)
    if has_res:
        y = y + hprev_ref[...]
    y_ref[...] = y
    if has_next:
        hn_ref[...] = dinv * jnp.dot(y, wn_ref[...],
                                     preferred_element_type=jnp.float32)


def _make_combine(has_res, has_next):
    outs = [jax.ShapeDtypeStruct((_N, _D), jnp.float32)]
    if has_next:
        outs.append(jax.ShapeDtypeStruct((_N, _D), jnp.float32))
    return pl.pallas_call(
        functools.partial(_combine_body, has_res, has_next),
        out_shape=outs,
    )


_combine_first = _make_combine(False, True)
_combine_mid = _make_combine(True, True)
_combine_last = _make_combine(True, False)


def kernel(x, edge_index, W1, b1, g1, be1, W2, b2, g2, be2, W3, b3, g3, be3):
    src3 = edge_index[0].reshape(_NW, _NCHUNK, _CH)
    dst3 = edge_index[1].reshape(_NW, _NCHUNK, _CH)

    b1r, g1r, be1r = b1.reshape(1, _D), g1.reshape(1, _D), be1.reshape(1, _D)
    b2r, g2r, be2r = b2.reshape(1, _D), g2.reshape(1, _D), be2.reshape(1, _D)
    b3r, g3r, be3r = b3.reshape(1, _D), g3.reshape(1, _D), be3.reshape(1, _D)

    zeros_idx = jnp.zeros_like(src3)
    srcs = jnp.stack([zeros_idx, src3, src3, src3])

    def _step(carry, xs):
        i, src_t = xs
        dinv, hp, hres = carry

        s = _scatter_kernel(hp, src_t, dst3)

        def _pass0(_):
            dinv2, hp2 = _prep_call(s, x, W1)
            return (dinv2, hp2, hres)

        def _pass1(_):
            h1, hp2 = _combine_first(s, hp, dinv, b1r, g1r, be1r, W2)
            return (dinv, hp2, h1)

        def _pass2(_):
            h2, hp3 = _combine_mid(s, hp, dinv, b2r, g2r, be2r, hres, W3)
            return (dinv, hp3, h2)

        def _pass3(_):
            (h3,) = _combine_last(s, hp, dinv, b3r, g3r, be3r, hres)
            return (dinv, hp, h3)

        new_carry = lax.switch(i, [_pass0, _pass1, _pass2, _pass3], None)
        return new_carry, None

    init = (
        jnp.zeros((_N, 1), jnp.float32),
        jnp.ones((_N, _D), jnp.float32),
        jnp.zeros((_N, _D), jnp.float32),
    )
    (_, _, h3), _ = lax.scan(_step, init,
                             (jnp.arange(4, dtype=jnp.int32), srcs))
    return h3


# speedup vs baseline: 48.9338x; 48.9338x over previous
import functools

import jax
import jax.numpy as jnp
from jax import lax
from jax.experimental import pallas as pl
from jax.experimental.pallas import tpu as pltpu
from jax.experimental.pallas import tpu_sc as plsc

_N = 10000
_E = 320000
_D = 128
_EPS = 1e-5

_NC = 2
_NS = 16
_NW = _NC * _NS
_EPT = _E // _NW
_CH = 80
_NCHUNK = _EPT // _CH
_GR = _CH // 16

_NPH = 4
_PH = 2560
_ND = 128
_ACC = _PH + _ND
_RZ = _ACC // _NS
_RW = _PH // _NS
_NP2 = _NPH * _PH

_mesh = plsc.VectorSubcoreMesh(core_axis_name="c", subcore_axis_name="s")


@functools.partial(
    pl.kernel,
    out_type=jax.ShapeDtypeStruct((_NC, _NP2, _D), jnp.float32),
    mesh=_mesh,
    scratch_types=[
        pltpu.VMEM((_NCHUNK, _CH), jnp.int32),
        pltpu.VMEM((_NCHUNK, _CH), jnp.int32),
        pltpu.VMEM((_NCHUNK, _CH), jnp.int32),
        pltpu.VMEM((_CH, _D), jnp.float32),
        pltpu.VMEM((_RZ, _D), jnp.float32),
        pltpu.VMEM_SHARED((_ACC, _D), jnp.float32),
        pltpu.SemaphoreType.DMA,
        pltpu.SemaphoreType.DMA,
    ],
)
def _scatter_kernel(hp_hbm, src_hbm, dst_hbm, out_hbm,
                    src_v, dst_v, dstp_v, rows_v, zbuf_v, acc_sh,
                    gsem, ssem):
    c = lax.axis_index("c")
    s = lax.axis_index("s")
    wid = c * _NS + s

    pltpu.sync_copy(src_hbm.at[wid], src_v)
    pltpu.sync_copy(dst_hbm.at[wid], dst_v)

    def _zbufz(r, _):
        for k in range(_D // 16):
            zbuf_v[r, pl.ds(k * 16, 16)] = jnp.zeros((16,), jnp.float32)
        return 0

    for phase in range(_NPH):
        lo = phase * _PH
        lax.fori_loop(0, _RZ, _zbufz, 0)
        pltpu.sync_copy(zbuf_v, acc_sh.at[pl.ds(s * _RZ, _RZ)])

        lo_i = jnp.int32(lo)
        hi_i = jnp.int32(lo + _PH)
        ph_i = jnp.int32(_PH)
        m_i = jnp.int32(_ND - 1)

        def _clamp(r, _):
            for k in range(_GR):
                d16 = dst_v[r, pl.ds(k * 16, 16)]
                ok = (d16 >= lo_i) & (d16 < hi_i)
                dstp_v[r, pl.ds(k * 16, 16)] = jnp.where(
                    ok, d16 - lo_i, ph_i + (d16 & m_i))
            return 0

        lax.fori_loop(0, _NCHUNK, _clamp, 0)
        plsc.subcore_barrier()

        def _edge_chunk(j, _):
            pltpu.async_copy(hp_hbm.at[src_v.at[j]], rows_v, gsem).wait()
            pltpu.async_copy(rows_v, acc_sh.at[dstp_v.at[j]], ssem,
                             add=True).wait()
            return 0

        lax.fori_loop(0, _NCHUNK, _edge_chunk, 0)
        plsc.subcore_barrier()

        pltpu.sync_copy(acc_sh.at[pl.ds(s * _RW, _RW)], zbuf_v.at[pl.ds(0, _RW)])
        pltpu.sync_copy(zbuf_v.at[pl.ds(0, _RW)],
                        out_hbm.at[c, pl.ds(lo + s * _RW, _RW)])
        plsc.subcore_barrier()


def _prep_body(degp_ref, x_ref, w1_ref, dinv_ref, hp_ref):
    deg = degp_ref[0, 0:_N, 0:1] + degp_ref[1, 0:_N, 0:1] + 1.0
    dinv = lax.rsqrt(deg)
    dinv_ref[...] = dinv
    h1 = jnp.dot(x_ref[...], w1_ref[...], preferred_element_type=jnp.float32)
    hp_ref[...] = dinv * h1


_prep_call = pl.pallas_call(
    _prep_body,
    out_shape=[
        jax.ShapeDtypeStruct((_N, 1), jnp.float32),
        jax.ShapeDtypeStruct((_N, _D), jnp.float32),
    ],
)


def _combine_body(has_res, has_next, *refs):
    if has_res and has_next:
        (s_ref, hp_ref, dinv_ref, b_ref, g_ref, be_ref,
         hprev_ref, wn_ref, y_ref, hn_ref) = refs
    elif has_next:
        (s_ref, hp_ref, dinv_ref, b_ref, g_ref, be_ref,
         wn_ref, y_ref, hn_ref) = refs
    else:
        (s_ref, hp_ref, dinv_ref, b_ref, g_ref, be_ref,
         hprev_ref, y_ref) = refs

    dinv = dinv_ref[...]
    agg = s_ref[0, 0:_N, :] + s_ref[1, 0:_N, :] + hp_ref[...]
    z = dinv * agg + b_ref[...]
    mu = jnp.mean(z, axis=0, keepdims=True)
    zc = z - mu
    var = jnp.mean(zc * zc, axis=0, keepdims=True)
    y = g_ref[...] * zc * lax.rsqrt(var + _EPS) + be_ref[...]
    y = jnp.maximum(y, 0.0)
    if has_res:
        y = y + hprev_ref[...]
    y_ref[...] = y
    if has_next:
        hn_ref[...] = dinv * jnp.dot(y, wn_ref[...],
                                     preferred_element_type=jnp.float32)


def _make_combine(has_res, has_next):
    outs = [jax.ShapeDtypeStruct((_N, _D), jnp.float32)]
    if has_next:
        outs.append(jax.ShapeDtypeStruct((_N, _D), jnp.float32))
    return pl.pallas_call(
        functools.partial(_combine_body, has_res, has_next),
        out_shape=outs,
    )


_combine_first = _make_combine(False, True)
_combine_mid = _make_combine(True, True)
_combine_last = _make_combine(True, False)


def kernel(x, edge_index, W1, b1, g1, be1, W2, b2, g2, be2, W3, b3, g3, be3):
    src3 = edge_index[0].reshape(_NW, _NCHUNK, _CH)
    dst3 = edge_index[1].reshape(_NW, _NCHUNK, _CH)

    b1r, g1r, be1r = b1.reshape(1, _D), g1.reshape(1, _D), be1.reshape(1, _D)
    b2r, g2r, be2r = b2.reshape(1, _D), g2.reshape(1, _D), be2.reshape(1, _D)
    b3r, g3r, be3r = b3.reshape(1, _D), g3.reshape(1, _D), be3.reshape(1, _D)

    def _step(carry, i):
        dinv, hp, hres = carry

        s = _scatter_kernel(hp, src3, dst3)

        def _pass0(_):
            dinv2, hp2 = _prep_call(s, x, W1)
            return (dinv2, hp2, hres)

        def _pass1(_):
            h1, hp2 = _combine_first(s, hp, dinv, b1r, g1r, be1r, W2)
            return (dinv, hp2, h1)

        def _pass2(_):
            h2, hp3 = _combine_mid(s, hp, dinv, b2r, g2r, be2r, hres, W3)
            return (dinv, hp3, h2)

        def _pass3(_):
            (h3,) = _combine_last(s, hp, dinv, b3r, g3r, be3r, hres)
            return (dinv, hp, h3)

        new_carry = lax.switch(i, [_pass0, _pass1, _pass2, _pass3], None)
        return new_carry, None

    init = (
        jnp.zeros((_N, 1), jnp.float32),
        jnp.ones((_N, _D), jnp.float32),
        jnp.zeros((_N, _D), jnp.float32),
    )
    (_, _, h3), _ = lax.scan(_step, init, jnp.arange(4, dtype=jnp.int32))
    return h3


# speedup vs baseline: 63.7759x; 1.3033x over previous
import functools

import jax
import jax.numpy as jnp
from jax import lax
from jax.experimental import pallas as pl
from jax.experimental.pallas import tpu as pltpu
from jax.experimental.pallas import tpu_sc as plsc

_N = 10000
_E = 320000
_D = 128
_EPS = 1e-5

_NC = 2
_NS = 16
_NW = _NC * _NS
_EPT = _E // _NW
_CH = 80
_NCHUNK = _EPT // _CH
_GR = _CH // 16

_NPH = 3
_PH = 4096
_ND = 128
_ACC = _PH + _ND
_RZ = _ACC // _NS
_RW = _PH // _NS
_NP2 = _NPH * _PH

_mesh = plsc.VectorSubcoreMesh(core_axis_name="c", subcore_axis_name="s")


@functools.partial(
    pl.kernel,
    out_type=jax.ShapeDtypeStruct((_NC, _NP2, _D), jnp.float32),
    mesh=_mesh,
    scratch_types=[
        pltpu.VMEM((_NCHUNK, _CH), jnp.int32),
        pltpu.VMEM((_NCHUNK, _CH), jnp.int32),
        pltpu.VMEM((_NCHUNK, _CH), jnp.int32),
        pltpu.VMEM((_CH, _D), jnp.float32),
        pltpu.VMEM((_RZ, _D), jnp.float32),
        pltpu.VMEM_SHARED((_ACC, _D), jnp.float32),
        pltpu.SemaphoreType.DMA,
        pltpu.SemaphoreType.DMA,
    ],
)
def _scatter_kernel(hp_hbm, src_hbm, dst_hbm, out_hbm,
                    src_v, dst_v, dstp_v, rows_v, zbuf_v, acc_sh,
                    gsem, ssem):
    c = lax.axis_index("c")
    s = lax.axis_index("s")
    wid = c * _NS + s

    pltpu.sync_copy(src_hbm.at[wid], src_v)
    pltpu.sync_copy(dst_hbm.at[wid], dst_v)

    def _zbufz(r, _):
        for k in range(_D // 16):
            zbuf_v[r, pl.ds(k * 16, 16)] = jnp.zeros((16,), jnp.float32)
        return 0

    for phase in range(_NPH):
        lo = phase * _PH
        lax.fori_loop(0, _RZ, _zbufz, 0)
        pltpu.sync_copy(zbuf_v, acc_sh.at[pl.ds(s * _RZ, _RZ)])

        lo_i = jnp.int32(lo)
        hi_i = jnp.int32(lo + _PH)
        ph_i = jnp.int32(_PH)
        m_i = jnp.int32(_ND - 1)

        def _clamp(r, _):
            for k in range(_GR):
                d16 = dst_v[r, pl.ds(k * 16, 16)]
                ok = (d16 >= lo_i) & (d16 < hi_i)
                dstp_v[r, pl.ds(k * 16, 16)] = jnp.where(
                    ok, d16 - lo_i, ph_i + (d16 & m_i))
            return 0

        lax.fori_loop(0, _NCHUNK, _clamp, 0)
        plsc.subcore_barrier()

        def _edge_chunk(j, _):
            pltpu.async_copy(hp_hbm.at[src_v.at[j]], rows_v, gsem).wait()
            pltpu.async_copy(rows_v, acc_sh.at[dstp_v.at[j]], ssem,
                             add=True).wait()
            return 0

        lax.fori_loop(0, _NCHUNK, _edge_chunk, 0)
        plsc.subcore_barrier()

        pltpu.sync_copy(acc_sh.at[pl.ds(s * _RW, _RW)], zbuf_v.at[pl.ds(0, _RW)])
        pltpu.sync_copy(zbuf_v.at[pl.ds(0, _RW)],
                        out_hbm.at[c, pl.ds(lo + s * _RW, _RW)])
        plsc.subcore_barrier()


def _prep_body(degp_ref, x_ref, w1_ref, dinv_ref, hp_ref):
    deg = degp_ref[0, 0:_N, 0:1] + degp_ref[1, 0:_N, 0:1] + 1.0
    dinv = lax.rsqrt(deg)
    dinv_ref[...] = dinv
    h1 = jnp.dot(x_ref[...], w1_ref[...], preferred_element_type=jnp.float32)
    hp_ref[...] = dinv * h1


_prep_call = pl.pallas_call(
    _prep_body,
    out_shape=[
        jax.ShapeDtypeStruct((_N, 1), jnp.float32),
        jax.ShapeDtypeStruct((_N, _D), jnp.float32),
    ],
)


def _combine_body(has_res, has_next, *refs):
    if has_res and has_next:
        (s_ref, hp_ref, dinv_ref, b_ref, g_ref, be_ref,
         hprev_ref, wn_ref, y_ref, hn_ref) = refs
    elif has_next:
        (s_ref, hp_ref, dinv_ref, b_ref, g_ref, be_ref,
         wn_ref, y_ref, hn_ref) = refs
    else:
        (s_ref, hp_ref, dinv_ref, b_ref, g_ref, be_ref,
         hprev_ref, y_ref) = refs

    dinv = dinv_ref[...]
    agg = s_ref[0, 0:_N, :] + s_ref[1, 0:_N, :] + hp_ref[...]
    z = dinv * agg + b_ref[...]
    mu = jnp.mean(z, axis=0, keepdims=True)
    zc = z - mu
    var = jnp.mean(zc * zc, axis=0, keepdims=True)
    y = g_ref[...] * zc * lax.rsqrt(var + _EPS) + be_ref[...]
    y = jnp.maximum(y, 0.0)
    if has_res:
        y = y + hprev_ref[...]
    y_ref[...] = y
    if has_next:
        hn_ref[...] = dinv * jnp.dot(y, wn_ref[...],
                                     preferred_element_type=jnp.float32)


def _make_combine(has_res, has_next):
    outs = [jax.ShapeDtypeStruct((_N, _D), jnp.float32)]
    if has_next:
        outs.append(jax.ShapeDtypeStruct((_N, _D), jnp.float32))
    return pl.pallas_call(
        functools.partial(_combine_body, has_res, has_next),
        out_shape=outs,
    )


_combine_first = _make_combine(False, True)
_combine_mid = _make_combine(True, True)
_combine_last = _make_combine(True, False)


def kernel(x, edge_index, W1, b1, g1, be1, W2, b2, g2, be2, W3, b3, g3, be3):
    src3 = edge_index[0].reshape(_NW, _NCHUNK, _CH)
    dst3 = edge_index[1].reshape(_NW, _NCHUNK, _CH)

    b1r, g1r, be1r = b1.reshape(1, _D), g1.reshape(1, _D), be1.reshape(1, _D)
    b2r, g2r, be2r = b2.reshape(1, _D), g2.reshape(1, _D), be2.reshape(1, _D)
    b3r, g3r, be3r = b3.reshape(1, _D), g3.reshape(1, _D), be3.reshape(1, _D)

    n_pass = jnp.int32(4) + lax.shift_right_logical(dst3[0, 0, 0],
                                                    jnp.int32(30))

    def _cond(st):
        return st[0] < n_pass

    def _step(st):
        i, dinv, hp, hres = st

        s = _scatter_kernel(hp, src3, dst3)

        def _pass0(_):
            dinv2, hp2 = _prep_call(s, x, W1)
            return (dinv2, hp2, hres)

        def _pass1(_):
            h1, hp2 = _combine_first(s, hp, dinv, b1r, g1r, be1r, W2)
            return (dinv, hp2, h1)

        def _pass2(_):
            h2, hp3 = _combine_mid(s, hp, dinv, b2r, g2r, be2r, hres, W3)
            return (dinv, hp3, h2)

        def _pass3(_):
            (h3,) = _combine_last(s, hp, dinv, b3r, g3r, be3r, hres)
            return (dinv, hp, h3)

        new_carry = lax.switch(i, [_pass0, _pass1, _pass2, _pass3], None)
        return (i + 1,) + new_carry

    init = (
        jnp.int32(0),
        jnp.zeros((_N, 1), jnp.float32),
        jnp.ones((_N, _D), jnp.float32),
        jnp.zeros((_N, _D), jnp.float32),
    )
    _, _, _, h3 = lax.while_loop(_cond, _step, init)
    return h3
